# Initial kernel scaffold; baseline (speedup 1.0000x reference)
import jax
import jax.numpy as jnp
from jax.experimental import pallas as pl


def kernel(abs_pos, vel_hist, rel_disp, rel_dist, senders, receivers, tag, vel_mean, vel_std, acc_mean, acc_std, params):
    raise NotImplementedError("write your pallas kernel here")


# speedup vs baseline: 3.5216x; 3.5216x over previous
import functools

import jax
import jax.numpy as jnp
from jax import lax
from jax.experimental import pallas as pl
from jax.experimental.pallas import tpu as pltpu
from jax.experimental.pallas import tpu_sc as plsc

N = 10000
E = 160000
DIM = 3
HIST = 6
LATENT = 128
NUM_TYPES = 9
DT = 0.0025

NB = 2000
EB = 2000
F32 = jnp.float32

_NC = 2
_NS = 16
_NW = _NC * _NS
_CH = 128
_NCHUNKS = E // _CH
_ROWS_PER_SUB = 624
_TAIL_BASE = _ROWS_PER_SUB * _NS
_TAIL = N - _TAIL_BASE


def _ln(x, scale, offset):
    m = jnp.mean(x, axis=-1, keepdims=True)
    v = jnp.mean((x - m) ** 2, axis=-1, keepdims=True)
    return (x - m) / jnp.sqrt(v + 1e-6) * scale + offset


def _dot(a, b):
    return jnp.dot(a, b, preferred_element_type=F32)


def _node_enc_body(vh_ref, tag_ref, w0v_ref, ew_ref, b0_ref, w1_ref, b1_ref,
                   w2_ref, b2_ref, sc_ref, of_ref, ws_ref, wr_ref,
                   nodes_ref, sp_ref, rp_ref):
    tag = tag_ref[0, 0, :]
    oh = (tag[:, None] == lax.broadcasted_iota(jnp.int32, (NB, NUM_TYPES), 1)
          ).astype(F32)
    h = _dot(vh_ref[...], w0v_ref[...]) + _dot(oh, ew_ref[...]) + b0_ref[...]
    h = jnp.maximum(h, 0.0)
    h = jnp.maximum(_dot(h, w1_ref[...]) + b1_ref[...], 0.0)
    h = _dot(h, w2_ref[...]) + b2_ref[...]
    n = _ln(h, sc_ref[...], of_ref[...])
    nodes_ref[...] = n
    sp_ref[...] = _dot(n, ws_ref[...])
    rp_ref[...] = _dot(n, wr_ref[...])


def _edge_enc_body(ef_ref, w0_ref, b0_ref, w1_ref, b1_ref, w2_ref, b2_ref,
                   sc_ref, of_ref, out_ref):
    h = jnp.maximum(_dot(ef_ref[...], w0_ref[...]) + b0_ref[...], 0.0)
    h = jnp.maximum(_dot(h, w1_ref[...]) + b1_ref[...], 0.0)
    h = _dot(h, w2_ref[...]) + b2_ref[...]
    out_ref[...] = _ln(h, sc_ref[...], of_ref[...])


def _edge_upd_body(e_ref, gs_ref, gr_ref, we_ref, b0_ref, w1_ref, b1_ref,
                   w2_ref, b2_ref, sc_ref, of_ref, out_ref):
    e = e_ref[...]
    h = _dot(e, we_ref[...]) + gs_ref[...] + gr_ref[...] + b0_ref[...]
    h = jnp.maximum(h, 0.0)
    h = jnp.maximum(_dot(h, w1_ref[...]) + b1_ref[...], 0.0)
    h = _dot(h, w2_ref[...]) + b2_ref[...]
    out_ref[...] = e + _ln(h, sc_ref[...], of_ref[...])


def _node_upd_body(n_ref, a0_ref, a1_ref, wn_ref, wa_ref, b0_ref, w1_ref,
                   b1_ref, w2_ref, b2_ref, sc_ref, of_ref, ws_ref, wr_ref,
                   out_ref, sp_ref, rp_ref, *, with_proj):
    n = n_ref[...]
    agg = a0_ref[...] + a1_ref[...]
    h = _dot(n, wn_ref[...]) + _dot(agg, wa_ref[...]) + b0_ref[...]
    h = jnp.maximum(h, 0.0)
    h = jnp.maximum(_dot(h, w1_ref[...]) + b1_ref[...], 0.0)
    h = _dot(h, w2_ref[...]) + b2_ref[...]
    nn = n + _ln(h, sc_ref[...], of_ref[...])
    out_ref[...] = nn
    if with_proj:
        sp_ref[...] = _dot(nn, ws_ref[...])
        rp_ref[...] = _dot(nn, wr_ref[...])


def _dec_body(n_ref, r0_ref, lv_ref, tag_ref, st_ref, w0_ref, b0_ref, w1_ref,
              b1_ref, w2_ref, b2_ref, out_ref):
    h = jnp.maximum(_dot(n_ref[...], w0_ref[...]) + b0_ref[...], 0.0)
    h = jnp.maximum(_dot(h, w1_ref[...]) + b1_ref[...], 0.0)
    acc = _dot(h, w2_ref[...]) + b2_ref[...]
    st = st_ref[...]
    vm, vs = st[0:1, 0:DIM], st[1:2, 0:DIM]
    am, asd = st[2:3, 0:DIM], st[3:4, 0:DIM]
    r0 = r0_ref[...]
    u0 = (lv_ref[...] * vs + vm) / DT
    a = (acc * asd + am) / (DT * DT)
    u = u0 + DT * a
    r = r0 + DT * u
    r = r - jnp.floor(r)
    tag2d = tag_ref[0, 0, :][:, None]
    wall = jnp.where(tag2d == 3, 1.0, 0.0)
    r = wall * r0 + (1.0 - wall) * (r - jnp.floor(r))
    d = r - r0
    d = d - jnp.round(d)
    u2 = d / DT
    a2 = (u2 - u0) / DT
    out_ref[...] = (a2 * (DT * DT) - am) / asd


def _full(shape):
    return pl.BlockSpec(shape, lambda i: tuple(0 for _ in shape))


def _rowblk(cols, rows=NB):
    return pl.BlockSpec((rows, cols), lambda i: (i, 0))


_TAG_SPEC = pl.BlockSpec((1, 1, NB), lambda i: (i, 0, 0))


def _w_specs(*shapes):
    return [_full(s) for s in shapes]


def _node_encode(vh, tag3, w0v, ew, b0, w1, b1, w2, b2, sc, of, ws, wr):
    grid = (N // NB,)
    out_shape = [jax.ShapeDtypeStruct((N, LATENT), F32)] * 3
    return pl.pallas_call(
        _node_enc_body,
        grid=grid,
        in_specs=[_rowblk((HIST - 1) * DIM), _TAG_SPEC] + _w_specs(
            w0v.shape, ew.shape, b0.shape, w1.shape, b1.shape, w2.shape,
            b2.shape, sc.shape, of.shape, ws.shape, wr.shape),
        out_specs=[_rowblk(LATENT)] * 3,
        out_shape=out_shape,
    )(vh, tag3, w0v, ew, b0, w1, b1, w2, b2, sc, of, ws, wr)


def _edge_encode(ef, w0, b0, w1, b1, w2, b2, sc, of):
    grid = (E // EB,)
    return pl.pallas_call(
        _edge_enc_body,
        grid=grid,
        in_specs=[_rowblk(DIM + 1, EB)] + _w_specs(
            w0.shape, b0.shape, w1.shape, b1.shape, w2.shape, b2.shape,
            sc.shape, of.shape),
        out_specs=_rowblk(LATENT, EB),
        out_shape=jax.ShapeDtypeStruct((E, LATENT), F32),
    )(ef, w0, b0, w1, b1, w2, b2, sc, of)


def _edge_update(e, gs, gr, we, b0, w1, b1, w2, b2, sc, of):
    grid = (E // EB,)
    return pl.pallas_call(
        _edge_upd_body,
        grid=grid,
        in_specs=[_rowblk(LATENT, EB)] * 3 + _w_specs(
            we.shape, b0.shape, w1.shape, b1.shape, w2.shape, b2.shape,
            sc.shape, of.shape),
        out_specs=_rowblk(LATENT, EB),
        out_shape=jax.ShapeDtypeStruct((E, LATENT), F32),
    )(e, gs, gr, we, b0, w1, b1, w2, b2, sc, of)


def _node_update(n, a0, a1, wn, wa, b0, w1, b1, w2, b2, sc, of, ws, wr,
                 with_proj):
    grid = (N // NB,)
    nout = 3 if with_proj else 1
    body = functools.partial(_node_upd_body, with_proj=with_proj)
    if not with_proj:
        def body(*refs):
            _node_upd_body(*refs[:14], refs[14], None, None, with_proj=False)
    res = pl.pallas_call(
        body,
        grid=grid,
        in_specs=[_rowblk(LATENT)] * 3 + _w_specs(
            wn.shape, wa.shape, b0.shape, w1.shape, b1.shape, w2.shape,
            b2.shape, sc.shape, of.shape, ws.shape, wr.shape),
        out_specs=[_rowblk(LATENT)] * nout,
        out_shape=[jax.ShapeDtypeStruct((N, LATENT), F32)] * nout,
    )(n, a0, a1, wn, wa, b0, w1, b1, w2, b2, sc, of, ws, wr)
    if with_proj:
        return res
    return res[0], None, None


def _decode(n, r0, lv, tag3, stats, w0, b0, w1, b1, w2, b2):
    grid = (N // NB,)
    return pl.pallas_call(
        _dec_body,
        grid=grid,
        in_specs=[_rowblk(LATENT), _rowblk(DIM), _rowblk(DIM), _TAG_SPEC]
        + _w_specs(stats.shape, w0.shape, b0.shape, w1.shape, b1.shape,
                   w2.shape, b2.shape),
        out_specs=_rowblk(DIM),
        out_shape=jax.ShapeDtypeStruct((N, DIM), F32),
    )(n, r0, lv, tag3, stats, w0, b0, w1, b1, w2, b2)


def _worker_range(wid):
    q, r = divmod(_NCHUNKS, _NW)
    start = wid * q + jnp.minimum(wid, r)
    cnt = q + (wid < r).astype(jnp.int32)
    return start, cnt


@functools.cache
def _sc_kernels():
    mesh = plsc.VectorSubcoreMesh(core_axis_name="c", subcore_axis_name="s",
                                  num_cores=_NC, num_subcores=_NS)

    @functools.partial(
        pl.kernel,
        out_type=[jax.ShapeDtypeStruct((E, LATENT), F32)] * 2,
        mesh=mesh,
        scratch_types=[
            pltpu.VMEM((_CH,), jnp.int32),
            pltpu.VMEM((_CH,), jnp.int32),
            pltpu.VMEM((_CH, LATENT), F32),
            pltpu.VMEM((_CH, LATENT), F32),
            pltpu.SemaphoreType.DMA,
            pltpu.SemaphoreType.DMA,
        ],
    )
    def gather(sp_hbm, rp_hbm, snd_hbm, rcv_hbm, gs_hbm, gr_hbm,
               idx_s, idx_r, rows_s, rows_r, sem_s, sem_r):
        wid = lax.axis_index("s") * _NC + lax.axis_index("c")
        start, cnt = _worker_range(wid)

        def body(j, carry):
            off = (start + j) * _CH
            pltpu.sync_copy(snd_hbm.at[pl.ds(off, _CH)], idx_s)
            pltpu.sync_copy(rcv_hbm.at[pl.ds(off, _CH)], idx_r)
            cp1 = pltpu.async_copy(sp_hbm.at[idx_s], rows_s, sem_s)
            cp2 = pltpu.async_copy(rp_hbm.at[idx_r], rows_r, sem_r)
            cp1.wait()
            cp2.wait()
            pltpu.sync_copy(rows_s, gs_hbm.at[pl.ds(off, _CH)])
            pltpu.sync_copy(rows_r, gr_hbm.at[pl.ds(off, _CH)])
            return carry

        lax.fori_loop(0, cnt, body, 0)

    @functools.partial(
        pl.kernel,
        out_type=jax.ShapeDtypeStruct((_NC, N, LATENT), F32),
        mesh=mesh,
        scratch_types=[
            pltpu.VMEM((_CH,), jnp.int32),
            pltpu.VMEM((_CH, LATENT), F32),
            pltpu.VMEM_SHARED((N, LATENT), F32),
        ],
    )
    def scatter(e_hbm, rcv_hbm, zz_hbm, out_hbm, idx_v, vals_v, acc_sh):
        c = lax.axis_index("c")
        s = lax.axis_index("s")
        wid = s * _NC + c
        rbase = s * _ROWS_PER_SUB
        pltpu.sync_copy(zz_hbm.at[pl.ds(rbase, _ROWS_PER_SUB)],
                        acc_sh.at[pl.ds(rbase, _ROWS_PER_SUB)])

        @pl.when(s == _NS - 1)
        def _():
            pltpu.sync_copy(zz_hbm.at[pl.ds(_TAIL_BASE, _TAIL)],
                            acc_sh.at[pl.ds(_TAIL_BASE, _TAIL)])

        plsc.subcore_barrier()
        start, cnt = _worker_range(wid)

        def body(j, carry):
            off = (start + j) * _CH
            pltpu.sync_copy(rcv_hbm.at[pl.ds(off, _CH)], idx_v)
            pltpu.sync_copy(e_hbm.at[pl.ds(off, _CH)], vals_v)
            pltpu.sync_copy(vals_v, acc_sh.at[idx_v], add=True)
            return carry

        lax.fori_loop(0, cnt, body, 0)
        plsc.subcore_barrier()
        pltpu.sync_copy(acc_sh.at[pl.ds(rbase, _ROWS_PER_SUB)],
                        out_hbm.at[c, pl.ds(rbase, _ROWS_PER_SUB)])

        @pl.when(s == _NS - 1)
        def _():
            pltpu.sync_copy(acc_sh.at[pl.ds(_TAIL_BASE, _TAIL)],
                            out_hbm.at[c, pl.ds(_TAIL_BASE, _TAIL)])

    return gather, scatter


def _sc_gather(sp, rp, snd, rcv):
    return _sc_kernels()[0](sp, rp, snd, rcv)


def _sc_scatter(edges, rcv, zz):
    return _sc_kernels()[1](edges, rcv, zz)


def kernel(abs_pos, vel_hist, rel_disp, rel_dist, senders, receivers, tag,
           vel_mean, vel_std, acc_mean, acc_std, params):
    snd = senders.astype(jnp.int32)
    rcv = receivers.astype(jnp.int32)
    tag3 = tag.astype(jnp.int32).reshape(N // NB, 1, NB)
    ef = jnp.concatenate([rel_disp, rel_dist], axis=-1)
    zz = jnp.zeros((N, LATENT), F32)

    def row(v):
        return v.reshape(1, -1)

    en = params["enc_node"]
    w0 = en[0]["w"]
    w0v, w0t = w0[: (HIST - 1) * DIM], w0[(HIST - 1) * DIM:]
    ew = params["embed"] @ w0t
    ee = params["enc_edge"]

    proc = params["proc"]

    def edge_w(t):
        l = proc[t]["edge_mlp"]
        w = l[0]["w"]
        return (w[:LATENT], w[LATENT:2 * LATENT], w[2 * LATENT:],
                row(l[0]["b"]), l[1]["w"], row(l[1]["b"]), l[2]["w"],
                row(l[2]["b"]), row(proc[t]["edge_ln"]["scale"]),
                row(proc[t]["edge_ln"]["offset"]))

    def node_w(t):
        l = proc[t]["node_mlp"]
        w = l[0]["w"]
        return (w[:LATENT], w[LATENT:], row(l[0]["b"]), l[1]["w"],
                row(l[1]["b"]), l[2]["w"], row(l[2]["b"]),
                row(proc[t]["node_ln"]["scale"]),
                row(proc[t]["node_ln"]["offset"]))

    we0, ws0, wr0 = edge_w(0)[:3]

    nodes, sp, rp = _node_encode(
        vel_hist, tag3, w0v, ew, row(en[0]["b"]), en[1]["w"], row(en[1]["b"]),
        en[2]["w"], row(en[2]["b"]), row(params["enc_node_ln"]["scale"]),
        row(params["enc_node_ln"]["offset"]), ws0, wr0)

    edges = _edge_encode(
        ef, ee[0]["w"], row(ee[0]["b"]), ee[1]["w"], row(ee[1]["b"]),
        ee[2]["w"], row(ee[2]["b"]), row(params["enc_edge_ln"]["scale"]),
        row(params["enc_edge_ln"]["offset"]))

    n_steps = len(proc)
    for t in range(n_steps):
        we, ws, wr, eb0, ew1, eb1, ew2, eb2, esc, eof = edge_w(t)
        gs, gr = _sc_gather(sp, rp, snd, rcv)
        edges = _edge_update(edges, gs, gr, we, eb0, ew1, eb1, ew2, eb2,
                             esc, eof)
        agg2 = _sc_scatter(edges, rcv, zz)
        wn, wa, nb0, nw1, nb1, nw2, nb2, nsc, nof = node_w(t)
        last = t == n_steps - 1
        ws_n, wr_n = (ws, wr) if last else edge_w(t + 1)[1:3]
        nodes, sp, rp = _node_update(
            nodes, agg2[0], agg2[1], wn, wa, nb0, nw1, nb1, nw2, nb2,
            nsc, nof, ws_n, wr_n, with_proj=not last)

    stats = jnp.zeros((8, LATENT), F32)
    stats = stats.at[0, :DIM].set(vel_mean).at[1, :DIM].set(vel_std)
    stats = stats.at[2, :DIM].set(acc_mean).at[3, :DIM].set(acc_std)
    dec = params["dec"]
    return _decode(
        nodes, abs_pos[:, -1], vel_hist[:, -DIM:], tag3, stats,
        dec[0]["w"], row(dec[0]["b"]), dec[1]["w"], row(dec[1]["b"]),
        dec[2]["w"], row(dec[2]["b"]))


# speedup vs baseline: 4.8924x; 1.3892x over previous
import functools

import jax
import jax.numpy as jnp
from jax import lax
from jax.experimental import pallas as pl
from jax.experimental.pallas import tpu as pltpu
from jax.experimental.pallas import tpu_sc as plsc

N = 10000
E = 160000
DIM = 3
HIST = 6
LATENT = 128
NUM_TYPES = 9
DT = 0.0025

NB = 2000
EB = 2000
F32 = jnp.float32

_NC = 2
_NS = 16
_NW = _NC * _NS
_CH = 128
_NCHUNKS = E // _CH
_ROWS_PER_SUB = 624
_TAIL_BASE = _ROWS_PER_SUB * _NS
_TAIL = N - _TAIL_BASE


def _ln(x, scale, offset):
    m = jnp.mean(x, axis=-1, keepdims=True)
    v = jnp.mean((x - m) ** 2, axis=-1, keepdims=True)
    return (x - m) / jnp.sqrt(v + 1e-6) * scale + offset


def _dot(a, b):
    return jnp.dot(a, b, preferred_element_type=F32)


def _node_enc_body(vh_ref, tag_ref, w0v_ref, ew_ref, b0_ref, w1_ref, b1_ref,
                   w2_ref, b2_ref, sc_ref, of_ref, ws_ref, wr_ref,
                   nodes_ref, sp_ref, rp_ref):
    tag = tag_ref[0, 0, :]
    oh = (tag[:, None] == lax.broadcasted_iota(jnp.int32, (NB, NUM_TYPES), 1)
          ).astype(F32)
    h = _dot(vh_ref[...], w0v_ref[...]) + _dot(oh, ew_ref[...]) + b0_ref[...]
    h = jnp.maximum(h, 0.0)
    h = jnp.maximum(_dot(h, w1_ref[...]) + b1_ref[...], 0.0)
    h = _dot(h, w2_ref[...]) + b2_ref[...]
    n = _ln(h, sc_ref[...], of_ref[...])
    nodes_ref[...] = n
    sp_ref[...] = _dot(n, ws_ref[...])
    rp_ref[...] = _dot(n, wr_ref[...])


def _edge_enc_body(ef_ref, w0_ref, b0_ref, w1_ref, b1_ref, w2_ref, b2_ref,
                   sc_ref, of_ref, out_ref):
    h = jnp.maximum(_dot(ef_ref[...], w0_ref[...]) + b0_ref[...], 0.0)
    h = jnp.maximum(_dot(h, w1_ref[...]) + b1_ref[...], 0.0)
    h = _dot(h, w2_ref[...]) + b2_ref[...]
    out_ref[...] = _ln(h, sc_ref[...], of_ref[...])


def _edge_upd_body(e_ref, g_ref, we_ref, b0_ref, w1_ref, b1_ref,
                   w2_ref, b2_ref, sc_ref, of_ref, out_ref):
    e = e_ref[...]
    h = _dot(e, we_ref[...]) + g_ref[...] + b0_ref[...]
    h = jnp.maximum(h, 0.0)
    h = jnp.maximum(_dot(h, w1_ref[...]) + b1_ref[...], 0.0)
    h = _dot(h, w2_ref[...]) + b2_ref[...]
    out_ref[...] = e + _ln(h, sc_ref[...], of_ref[...])


def _node_upd_body(n_ref, a0_ref, a1_ref, wn_ref, wa_ref, b0_ref, w1_ref,
                   b1_ref, w2_ref, b2_ref, sc_ref, of_ref, ws_ref, wr_ref,
                   out_ref, sp_ref, rp_ref, *, with_proj):
    n = n_ref[...]
    agg = a0_ref[...] + a1_ref[...]
    h = _dot(n, wn_ref[...]) + _dot(agg, wa_ref[...]) + b0_ref[...]
    h = jnp.maximum(h, 0.0)
    h = jnp.maximum(_dot(h, w1_ref[...]) + b1_ref[...], 0.0)
    h = _dot(h, w2_ref[...]) + b2_ref[...]
    nn = n + _ln(h, sc_ref[...], of_ref[...])
    out_ref[...] = nn
    if with_proj:
        sp_ref[...] = _dot(nn, ws_ref[...])
        rp_ref[...] = _dot(nn, wr_ref[...])


def _dec_body(n_ref, r0_ref, lv_ref, tag_ref, st_ref, w0_ref, b0_ref, w1_ref,
              b1_ref, w2_ref, b2_ref, out_ref):
    h = jnp.maximum(_dot(n_ref[...], w0_ref[...]) + b0_ref[...], 0.0)
    h = jnp.maximum(_dot(h, w1_ref[...]) + b1_ref[...], 0.0)
    acc = _dot(h, w2_ref[...]) + b2_ref[...]
    st = st_ref[...]
    vm, vs = st[0:1, 0:DIM], st[1:2, 0:DIM]
    am, asd = st[2:3, 0:DIM], st[3:4, 0:DIM]
    r0 = r0_ref[...]
    u0 = (lv_ref[...] * vs + vm) / DT
    a = (acc * asd + am) / (DT * DT)
    u = u0 + DT * a
    r = r0 + DT * u
    r = r - jnp.floor(r)
    tag2d = tag_ref[0, 0, :][:, None]
    wall = jnp.where(tag2d == 3, 1.0, 0.0)
    r = wall * r0 + (1.0 - wall) * (r - jnp.floor(r))
    d = r - r0
    d = d - jnp.round(d)
    u2 = d / DT
    a2 = (u2 - u0) / DT
    out_ref[...] = (a2 * (DT * DT) - am) / asd


def _full(shape):
    return pl.BlockSpec(shape, lambda i: tuple(0 for _ in shape))


def _rowblk(cols, rows=NB):
    return pl.BlockSpec((rows, cols), lambda i: (i, 0))


_TAG_SPEC = pl.BlockSpec((1, 1, NB), lambda i: (i, 0, 0))


def _w_specs(*shapes):
    return [_full(s) for s in shapes]


def _node_encode(vh, tag3, w0v, ew, b0, w1, b1, w2, b2, sc, of, ws, wr):
    grid = (N // NB,)
    out_shape = [jax.ShapeDtypeStruct((N, LATENT), F32)] * 3
    return pl.pallas_call(
        _node_enc_body,
        grid=grid,
        in_specs=[_rowblk((HIST - 1) * DIM), _TAG_SPEC] + _w_specs(
            w0v.shape, ew.shape, b0.shape, w1.shape, b1.shape, w2.shape,
            b2.shape, sc.shape, of.shape, ws.shape, wr.shape),
        out_specs=[_rowblk(LATENT)] * 3,
        out_shape=out_shape,
    )(vh, tag3, w0v, ew, b0, w1, b1, w2, b2, sc, of, ws, wr)


def _edge_encode(ef, w0, b0, w1, b1, w2, b2, sc, of):
    grid = (E // EB,)
    return pl.pallas_call(
        _edge_enc_body,
        grid=grid,
        in_specs=[_rowblk(DIM + 1, EB)] + _w_specs(
            w0.shape, b0.shape, w1.shape, b1.shape, w2.shape, b2.shape,
            sc.shape, of.shape),
        out_specs=_rowblk(LATENT, EB),
        out_shape=jax.ShapeDtypeStruct((E, LATENT), F32),
    )(ef, w0, b0, w1, b1, w2, b2, sc, of)


def _edge_update(e, g, we, b0, w1, b1, w2, b2, sc, of):
    grid = (E // EB,)
    return pl.pallas_call(
        _edge_upd_body,
        grid=grid,
        in_specs=[_rowblk(LATENT, EB)] * 2 + _w_specs(
            we.shape, b0.shape, w1.shape, b1.shape, w2.shape, b2.shape,
            sc.shape, of.shape),
        out_specs=_rowblk(LATENT, EB),
        out_shape=jax.ShapeDtypeStruct((E, LATENT), F32),
    )(e, g, we, b0, w1, b1, w2, b2, sc, of)


def _node_update(n, a0, a1, wn, wa, b0, w1, b1, w2, b2, sc, of, ws, wr,
                 with_proj):
    grid = (N // NB,)
    nout = 3 if with_proj else 1
    body = functools.partial(_node_upd_body, with_proj=with_proj)
    if not with_proj:
        def body(*refs):
            _node_upd_body(*refs[:14], refs[14], None, None, with_proj=False)
    res = pl.pallas_call(
        body,
        grid=grid,
        in_specs=[_rowblk(LATENT)] * 3 + _w_specs(
            wn.shape, wa.shape, b0.shape, w1.shape, b1.shape, w2.shape,
            b2.shape, sc.shape, of.shape, ws.shape, wr.shape),
        out_specs=[_rowblk(LATENT)] * nout,
        out_shape=[jax.ShapeDtypeStruct((N, LATENT), F32)] * nout,
    )(n, a0, a1, wn, wa, b0, w1, b1, w2, b2, sc, of, ws, wr)
    if with_proj:
        return res
    return res[0], None, None


def _decode(n, r0, lv, tag3, stats, w0, b0, w1, b1, w2, b2):
    grid = (N // NB,)
    return pl.pallas_call(
        _dec_body,
        grid=grid,
        in_specs=[_rowblk(LATENT), _rowblk(DIM), _rowblk(DIM), _TAG_SPEC]
        + _w_specs(stats.shape, w0.shape, b0.shape, w1.shape, b1.shape,
                   w2.shape, b2.shape),
        out_specs=_rowblk(DIM),
        out_shape=jax.ShapeDtypeStruct((N, DIM), F32),
    )(n, r0, lv, tag3, stats, w0, b0, w1, b1, w2, b2)


_CPW = 40
_PAD_CHUNKS = _CPW * _NW


def _worker_range(wid):
    start = wid * _CPW
    cnt = jnp.clip(_NCHUNKS - start, 0, _CPW)
    return start, cnt


@functools.cache
def _sc_kernels():
    mesh = plsc.VectorSubcoreMesh(core_axis_name="c", subcore_axis_name="s",
                                  num_cores=_NC, num_subcores=_NS)

    @functools.partial(
        pl.kernel,
        out_type=jax.ShapeDtypeStruct((E, LATENT), F32),
        mesh=mesh,
        scratch_types=[
            pltpu.VMEM((_CPW, _CH), jnp.int32),
            pltpu.VMEM((_CPW, _CH), jnp.int32),
            pltpu.VMEM((_CH, LATENT), F32),
            pltpu.VMEM((_CH, LATENT), F32),
            pltpu.VMEM((_CH, LATENT), F32),
            pltpu.VMEM((_CH, LATENT), F32),
            pltpu.VMEM((_CH, LATENT), F32),
            pltpu.VMEM((_CH, LATENT), F32),
            pltpu.SemaphoreType.DMA,
            pltpu.SemaphoreType.DMA,
            pltpu.SemaphoreType.DMA,
            pltpu.SemaphoreType.DMA,
        ],
    )
    def gather(sp_hbm, rp_hbm, snd2_hbm, rcv2_hbm, g_hbm,
               idx_s, idx_r, rs0, rs1, rr0, rr1, ob0, ob1,
               sg0, sg1, sw0, sw1):
        wid = lax.axis_index("s") * _NC + lax.axis_index("c")
        start, cnt = _worker_range(wid)
        rs = (rs0, rs1)
        rr = (rr0, rr1)
        ob = (ob0, ob1)
        sg = (sg0, sg1)
        sw = (sw0, sw1)

        pltpu.sync_copy(snd2_hbm.at[pl.ds(start, _CPW)], idx_s)
        pltpu.sync_copy(rcv2_hbm.at[pl.ds(start, _CPW)], idx_r)

        def fire(j, b):
            pltpu.async_copy(sp_hbm.at[idx_s.at[j]], rs[b], sg[b])
            pltpu.async_copy(rp_hbm.at[idx_r.at[j]], rr[b], sg[b])

        fire(0, 0)
        fire(1, 1)

        @pl.loop(0, cnt)
        def _(j):
            for b in range(2):
                @pl.when(lax.rem(j, 2) == b)
                def _():
                    @pl.when(j >= 2)
                    def _():
                        pltpu.make_async_copy(
                            g_hbm.at[pl.ds(0, _CH)], ob[b], sw[b]).wait()
                    pltpu.make_async_copy(
                        sp_hbm.at[pl.ds(0, _CH)], rs[b], sg[b]).wait()
                    pltpu.make_async_copy(
                        sp_hbm.at[pl.ds(0, _CH)], rr[b], sg[b]).wait()

                    @pl.loop(0, _CH)
                    def _(r):
                        for k in range(LATENT // 16):
                            sl = pl.ds(k * 16, 16)
                            ob[b][r, sl] = rs[b][r, sl] + rr[b][r, sl]

                    off = (start + j) * _CH
                    pltpu.async_copy(ob[b], g_hbm.at[pl.ds(off, _CH)], sw[b])

                    @pl.when(j + 2 < cnt)
                    def _():
                        fire(j + 2, b)

        pltpu.make_async_copy(g_hbm.at[pl.ds(0, _CH)], ob0, sw0).wait()
        pltpu.make_async_copy(g_hbm.at[pl.ds(0, _CH)], ob1, sw1).wait()

    @functools.partial(
        pl.kernel,
        out_type=jax.ShapeDtypeStruct((_NC, N, LATENT), F32),
        mesh=mesh,
        scratch_types=[
            pltpu.VMEM((_CPW, _CH), jnp.int32),
            pltpu.VMEM((_CH, LATENT), F32),
            pltpu.VMEM((_CH, LATENT), F32),
            pltpu.VMEM_SHARED((N, LATENT), F32),
            pltpu.SemaphoreType.DMA,
            pltpu.SemaphoreType.DMA,
        ],
    )
    def scatter(e_hbm, rcv2_hbm, zz_hbm, out_hbm, idx_v, v0, v1, acc_sh,
                sv0, sv1):
        c = lax.axis_index("c")
        s = lax.axis_index("s")
        wid = s * _NC + c
        start, cnt = _worker_range(wid)
        vals = (v0, v1)
        sv = (sv0, sv1)
        pltpu.sync_copy(rcv2_hbm.at[pl.ds(start, _CPW)], idx_v)
        rbase = s * _ROWS_PER_SUB
        pltpu.sync_copy(zz_hbm.at[pl.ds(rbase, _ROWS_PER_SUB)],
                        acc_sh.at[pl.ds(rbase, _ROWS_PER_SUB)])

        @pl.when(s == _NS - 1)
        def _():
            pltpu.sync_copy(zz_hbm.at[pl.ds(_TAIL_BASE, _TAIL)],
                            acc_sh.at[pl.ds(_TAIL_BASE, _TAIL)])

        def fire(j, b):
            pltpu.async_copy(e_hbm.at[pl.ds((start + j) * _CH, _CH)],
                             vals[b], sv[b])

        fire(0, 0)
        fire(1, 1)
        plsc.subcore_barrier()

        @pl.loop(0, cnt)
        def _(j):
            for b in range(2):
                @pl.when(lax.rem(j, 2) == b)
                def _():
                    pltpu.make_async_copy(
                        e_hbm.at[pl.ds(0, _CH)], vals[b], sv[b]).wait()
                    pltpu.sync_copy(vals[b], acc_sh.at[idx_v.at[j]], add=True)

                    @pl.when(j + 2 < cnt)
                    def _():
                        fire(j + 2, b)

        plsc.subcore_barrier()
        pltpu.sync_copy(acc_sh.at[pl.ds(rbase, _ROWS_PER_SUB)],
                        out_hbm.at[c, pl.ds(rbase, _ROWS_PER_SUB)])

        @pl.when(s == _NS - 1)
        def _():
            pltpu.sync_copy(acc_sh.at[pl.ds(_TAIL_BASE, _TAIL)],
                            out_hbm.at[c, pl.ds(_TAIL_BASE, _TAIL)])

    return gather, scatter


def _sc_gather(sp, rp, snd2, rcv2):
    return _sc_kernels()[0](sp, rp, snd2, rcv2)


def _sc_scatter(edges, rcv2, zz):
    return _sc_kernels()[1](edges, rcv2, zz)


def _pad2d(idx):
    idx2 = idx.reshape(_NCHUNKS, _CH)
    pad = jnp.zeros((_PAD_CHUNKS - _NCHUNKS, _CH), jnp.int32)
    return jnp.concatenate([idx2, pad], axis=0)


def kernel(abs_pos, vel_hist, rel_disp, rel_dist, senders, receivers, tag,
           vel_mean, vel_std, acc_mean, acc_std, params):
    snd2 = _pad2d(senders.astype(jnp.int32))
    rcv2 = _pad2d(receivers.astype(jnp.int32))
    tag3 = tag.astype(jnp.int32).reshape(N // NB, 1, NB)
    ef = jnp.concatenate([rel_disp, rel_dist], axis=-1)
    zz = jnp.zeros((N, LATENT), F32)

    def row(v):
        return v.reshape(1, -1)

    en = params["enc_node"]
    w0 = en[0]["w"]
    w0v, w0t = w0[: (HIST - 1) * DIM], w0[(HIST - 1) * DIM:]
    ew = params["embed"] @ w0t
    ee = params["enc_edge"]

    proc = params["proc"]

    def edge_w(t):
        l = proc[t]["edge_mlp"]
        w = l[0]["w"]
        return (w[:LATENT], w[LATENT:2 * LATENT], w[2 * LATENT:],
                row(l[0]["b"]), l[1]["w"], row(l[1]["b"]), l[2]["w"],
                row(l[2]["b"]), row(proc[t]["edge_ln"]["scale"]),
                row(proc[t]["edge_ln"]["offset"]))

    def node_w(t):
        l = proc[t]["node_mlp"]
        w = l[0]["w"]
        return (w[:LATENT], w[LATENT:], row(l[0]["b"]), l[1]["w"],
                row(l[1]["b"]), l[2]["w"], row(l[2]["b"]),
                row(proc[t]["node_ln"]["scale"]),
                row(proc[t]["node_ln"]["offset"]))

    we0, ws0, wr0 = edge_w(0)[:3]

    nodes, sp, rp = _node_encode(
        vel_hist, tag3, w0v, ew, row(en[0]["b"]), en[1]["w"], row(en[1]["b"]),
        en[2]["w"], row(en[2]["b"]), row(params["enc_node_ln"]["scale"]),
        row(params["enc_node_ln"]["offset"]), ws0, wr0)

    edges = _edge_encode(
        ef, ee[0]["w"], row(ee[0]["b"]), ee[1]["w"], row(ee[1]["b"]),
        ee[2]["w"], row(ee[2]["b"]), row(params["enc_edge_ln"]["scale"]),
        row(params["enc_edge_ln"]["offset"]))

    n_steps = len(proc)
    for t in range(n_steps):
        we, ws, wr, eb0, ew1, eb1, ew2, eb2, esc, eof = edge_w(t)
        g = _sc_gather(sp, rp, snd2, rcv2)
        edges = _edge_update(edges, g, we, eb0, ew1, eb1, ew2, eb2,
                             esc, eof)
        agg2 = _sc_scatter(edges, rcv2, zz)
        wn, wa, nb0, nw1, nb1, nw2, nb2, nsc, nof = node_w(t)
        last = t == n_steps - 1
        ws_n, wr_n = (ws, wr) if last else edge_w(t + 1)[1:3]
        nodes, sp, rp = _node_update(
            nodes, agg2[0], agg2[1], wn, wa, nb0, nw1, nb1, nw2, nb2,
            nsc, nof, ws_n, wr_n, with_proj=not last)

    stats = jnp.zeros((8, LATENT), F32)
    stats = stats.at[0, :DIM].set(vel_mean).at[1, :DIM].set(vel_std)
    stats = stats.at[2, :DIM].set(acc_mean).at[3, :DIM].set(acc_std)
    dec = params["dec"]
    return _decode(
        nodes, abs_pos[:, -1], vel_hist[:, -DIM:], tag3, stats,
        dec[0]["w"], row(dec[0]["b"]), dec[1]["w"], row(dec[1]["b"]),
        dec[2]["w"], row(dec[2]["b"]))


# speedup vs baseline: 5.2161x; 1.0662x over previous
import functools

import jax
import jax.numpy as jnp
from jax import lax
from jax.experimental import pallas as pl
from jax.experimental.pallas import tpu as pltpu
from jax.experimental.pallas import tpu_sc as plsc

N = 10000
E = 160000
DIM = 3
HIST = 6
LATENT = 128
NUM_TYPES = 9
DT = 0.0025

NB = 2000
EB = 2000
F32 = jnp.float32

_NC = 2
_NS = 16
_NW = _NC * _NS
_CH = 128
_NCHUNKS = E // _CH
_ROWS_PER_SUB = 624
_TAIL_BASE = _ROWS_PER_SUB * _NS
_TAIL = N - _TAIL_BASE


def _ln(x, scale, offset):
    m = jnp.mean(x, axis=-1, keepdims=True)
    v = jnp.mean((x - m) ** 2, axis=-1, keepdims=True)
    return (x - m) / jnp.sqrt(v + 1e-6) * scale + offset


def _dot(a, b):
    return jnp.dot(a, b, preferred_element_type=F32)


def _node_enc_body(vh_ref, tag_ref, w0v_ref, ew_ref, b0_ref, w1_ref, b1_ref,
                   w2_ref, b2_ref, sc_ref, of_ref, ws_ref, wr_ref,
                   nodes_ref, sp_ref, rp_ref):
    tag = tag_ref[0, 0, :]
    oh = (tag[:, None] == lax.broadcasted_iota(jnp.int32, (NB, NUM_TYPES), 1)
          ).astype(F32)
    h = _dot(vh_ref[...], w0v_ref[...]) + _dot(oh, ew_ref[...]) + b0_ref[...]
    h = jnp.maximum(h, 0.0)
    h = jnp.maximum(_dot(h, w1_ref[...]) + b1_ref[...], 0.0)
    h = _dot(h, w2_ref[...]) + b2_ref[...]
    n = _ln(h, sc_ref[...], of_ref[...])
    nodes_ref[...] = n
    sp_ref[...] = _dot(n, ws_ref[...])
    rp_ref[...] = _dot(n, wr_ref[...])


def _edge_enc_body(ef_ref, w0_ref, b0_ref, w1_ref, b1_ref, w2_ref, b2_ref,
                   sc_ref, of_ref, out_ref):
    h = jnp.maximum(_dot(ef_ref[...], w0_ref[...]) + b0_ref[...], 0.0)
    h = jnp.maximum(_dot(h, w1_ref[...]) + b1_ref[...], 0.0)
    h = _dot(h, w2_ref[...]) + b2_ref[...]
    out_ref[...] = _ln(h, sc_ref[...], of_ref[...])


def _edge_upd_body(e_ref, g_ref, we_ref, b0_ref, w1_ref, b1_ref,
                   w2_ref, b2_ref, sc_ref, of_ref, out_ref):
    e = e_ref[...]
    h = _dot(e, we_ref[...]) + g_ref[...] + b0_ref[...]
    h = jnp.maximum(h, 0.0)
    h = jnp.maximum(_dot(h, w1_ref[...]) + b1_ref[...], 0.0)
    h = _dot(h, w2_ref[...]) + b2_ref[...]
    out_ref[...] = e + _ln(h, sc_ref[...], of_ref[...])


def _node_upd_body(n_ref, a0_ref, a1_ref, a2_ref, a3_ref, wn_ref, wa_ref,
                   b0_ref, w1_ref, b1_ref, w2_ref, b2_ref, sc_ref, of_ref,
                   ws_ref, wr_ref, out_ref, sp_ref, rp_ref, *, with_proj):
    n = n_ref[...]
    agg = (a0_ref[...] + a1_ref[...]) + (a2_ref[...] + a3_ref[...])
    h = _dot(n, wn_ref[...]) + _dot(agg, wa_ref[...]) + b0_ref[...]
    h = jnp.maximum(h, 0.0)
    h = jnp.maximum(_dot(h, w1_ref[...]) + b1_ref[...], 0.0)
    h = _dot(h, w2_ref[...]) + b2_ref[...]
    nn = n + _ln(h, sc_ref[...], of_ref[...])
    out_ref[...] = nn
    if with_proj:
        sp_ref[...] = _dot(nn, ws_ref[...])
        rp_ref[...] = _dot(nn, wr_ref[...])


def _dec_body(n_ref, r0_ref, lv_ref, tag_ref, st_ref, w0_ref, b0_ref, w1_ref,
              b1_ref, w2_ref, b2_ref, out_ref):
    h = jnp.maximum(_dot(n_ref[...], w0_ref[...]) + b0_ref[...], 0.0)
    h = jnp.maximum(_dot(h, w1_ref[...]) + b1_ref[...], 0.0)
    acc = _dot(h, w2_ref[...]) + b2_ref[...]
    st = st_ref[...]
    vm, vs = st[0:1, 0:DIM], st[1:2, 0:DIM]
    am, asd = st[2:3, 0:DIM], st[3:4, 0:DIM]
    r0 = r0_ref[...]
    u0 = (lv_ref[...] * vs + vm) / DT
    a = (acc * asd + am) / (DT * DT)
    u = u0 + DT * a
    r = r0 + DT * u
    r = r - jnp.floor(r)
    tag2d = tag_ref[0, 0, :][:, None]
    wall = jnp.where(tag2d == 3, 1.0, 0.0)
    r = wall * r0 + (1.0 - wall) * (r - jnp.floor(r))
    d = r - r0
    d = d - jnp.round(d)
    u2 = d / DT
    a2 = (u2 - u0) / DT
    out_ref[...] = (a2 * (DT * DT) - am) / asd


def _full(shape):
    return pl.BlockSpec(shape, lambda i: tuple(0 for _ in shape))


def _rowblk(cols, rows=NB):
    return pl.BlockSpec((rows, cols), lambda i: (i, 0))


_TAG_SPEC = pl.BlockSpec((1, 1, NB), lambda i: (i, 0, 0))


def _w_specs(*shapes):
    return [_full(s) for s in shapes]


def _node_encode(vh, tag3, w0v, ew, b0, w1, b1, w2, b2, sc, of, ws, wr):
    grid = (N // NB,)
    out_shape = [jax.ShapeDtypeStruct((N, LATENT), F32)] * 3
    return pl.pallas_call(
        _node_enc_body,
        grid=grid,
        in_specs=[_rowblk((HIST - 1) * DIM), _TAG_SPEC] + _w_specs(
            w0v.shape, ew.shape, b0.shape, w1.shape, b1.shape, w2.shape,
            b2.shape, sc.shape, of.shape, ws.shape, wr.shape),
        out_specs=[_rowblk(LATENT)] * 3,
        out_shape=out_shape,
    )(vh, tag3, w0v, ew, b0, w1, b1, w2, b2, sc, of, ws, wr)


def _edge_encode(ef, w0, b0, w1, b1, w2, b2, sc, of):
    grid = (_EH // EB,)
    return pl.pallas_call(
        _edge_enc_body,
        grid=grid,
        in_specs=[_rowblk(DIM + 1, EB)] + _w_specs(
            w0.shape, b0.shape, w1.shape, b1.shape, w2.shape, b2.shape,
            sc.shape, of.shape),
        out_specs=_rowblk(LATENT, EB),
        out_shape=jax.ShapeDtypeStruct((_EH, LATENT), F32),
    )(ef, w0, b0, w1, b1, w2, b2, sc, of)


def _edge_update(e, g, we, b0, w1, b1, w2, b2, sc, of):
    grid = (_EH // EB,)
    return pl.pallas_call(
        _edge_upd_body,
        grid=grid,
        in_specs=[_rowblk(LATENT, EB)] * 2 + _w_specs(
            we.shape, b0.shape, w1.shape, b1.shape, w2.shape, b2.shape,
            sc.shape, of.shape),
        out_specs=_rowblk(LATENT, EB),
        out_shape=jax.ShapeDtypeStruct((_EH, LATENT), F32),
    )(e, g, we, b0, w1, b1, w2, b2, sc, of)


def _node_update(n, a0, a1, a2, a3, wn, wa, b0, w1, b1, w2, b2, sc, of,
                 ws, wr, with_proj):
    grid = (N // NB,)
    nout = 3 if with_proj else 1
    body = functools.partial(_node_upd_body, with_proj=with_proj)
    if not with_proj:
        def body(*refs):
            _node_upd_body(*refs[:16], refs[16], None, None, with_proj=False)
    res = pl.pallas_call(
        body,
        grid=grid,
        in_specs=[_rowblk(LATENT)] * 5 + _w_specs(
            wn.shape, wa.shape, b0.shape, w1.shape, b1.shape, w2.shape,
            b2.shape, sc.shape, of.shape, ws.shape, wr.shape),
        out_specs=[_rowblk(LATENT)] * nout,
        out_shape=[jax.ShapeDtypeStruct((N, LATENT), F32)] * nout,
    )(n, a0, a1, a2, a3, wn, wa, b0, w1, b1, w2, b2, sc, of, ws, wr)
    if with_proj:
        return res
    return res[0], None, None


def _decode(n, r0, lv, tag3, stats, w0, b0, w1, b1, w2, b2):
    grid = (N // NB,)
    return pl.pallas_call(
        _dec_body,
        grid=grid,
        in_specs=[_rowblk(LATENT), _rowblk(DIM), _rowblk(DIM), _TAG_SPEC]
        + _w_specs(stats.shape, w0.shape, b0.shape, w1.shape, b1.shape,
                   w2.shape, b2.shape),
        out_specs=_rowblk(DIM),
        out_shape=jax.ShapeDtypeStruct((N, DIM), F32),
    )(n, r0, lv, tag3, stats, w0, b0, w1, b1, w2, b2)


_EH = E // 2
_NCH_H = _EH // _CH
_PAD_CHUNKS = 640
_IDXROWS = 32


def _worker_range(wid):
    q, r = divmod(_NCH_H, _NW)
    start = wid * q + jnp.minimum(wid, r)
    cnt = q + (wid < r).astype(jnp.int32)
    return start, cnt


@functools.cache
def _sc_kernels():
    mesh = plsc.VectorSubcoreMesh(core_axis_name="c", subcore_axis_name="s",
                                  num_cores=_NC, num_subcores=_NS)

    @functools.partial(
        pl.kernel,
        out_type=jax.ShapeDtypeStruct((_EH, LATENT), F32),
        mesh=mesh,
        scratch_types=[
            pltpu.VMEM((_IDXROWS, _CH), jnp.int32),
            pltpu.VMEM((_IDXROWS, _CH), jnp.int32),
            pltpu.VMEM((_CH, LATENT), F32),
            pltpu.VMEM((_CH, LATENT), F32),
            pltpu.VMEM((_CH, LATENT), F32),
            pltpu.VMEM((_CH, LATENT), F32),
            pltpu.VMEM((_CH, LATENT), F32),
            pltpu.VMEM((_CH, LATENT), F32),
            pltpu.SemaphoreType.DMA,
            pltpu.SemaphoreType.DMA,
            pltpu.SemaphoreType.DMA,
            pltpu.SemaphoreType.DMA,
        ],
    )
    def gather(sp_hbm, rp_hbm, snd2_hbm, rcv2_hbm, g_hbm,
               idx_s, idx_r, rs0, rs1, rr0, rr1, ob0, ob1,
               sg0, sg1, sw0, sw1):
        wid = lax.axis_index("s") * _NC + lax.axis_index("c")
        start, cnt = _worker_range(wid)
        base = (start // 8) * 8
        boff = start - base
        rs = (rs0, rs1)
        rr = (rr0, rr1)
        ob = (ob0, ob1)
        sg = (sg0, sg1)
        sw = (sw0, sw1)

        pltpu.sync_copy(snd2_hbm.at[pl.ds(base, _IDXROWS)], idx_s)
        pltpu.sync_copy(rcv2_hbm.at[pl.ds(base, _IDXROWS)], idx_r)

        def fire(j, b):
            pltpu.async_copy(sp_hbm.at[idx_s.at[boff + j]], rs[b], sg[b])
            pltpu.async_copy(rp_hbm.at[idx_r.at[boff + j]], rr[b], sg[b])

        fire(0, 0)
        fire(1, 1)

        @pl.loop(0, cnt)
        def _(j):
            for b in range(2):
                @pl.when(lax.rem(j, 2) == b)
                def _():
                    @pl.when(j >= 2)
                    def _():
                        pltpu.make_async_copy(
                            g_hbm.at[pl.ds(0, _CH)], ob[b], sw[b]).wait()
                    pltpu.make_async_copy(
                        sp_hbm.at[pl.ds(0, _CH)], rs[b], sg[b]).wait()
                    pltpu.make_async_copy(
                        sp_hbm.at[pl.ds(0, _CH)], rr[b], sg[b]).wait()

                    @pl.loop(0, _CH)
                    def _(r):
                        for k in range(LATENT // 16):
                            sl = pl.ds(k * 16, 16)
                            ob[b][r, sl] = rs[b][r, sl] + rr[b][r, sl]

                    off = (start + j) * _CH
                    pltpu.async_copy(ob[b], g_hbm.at[pl.ds(off, _CH)], sw[b])

                    @pl.when(j + 2 < cnt)
                    def _():
                        fire(j + 2, b)

        pltpu.make_async_copy(g_hbm.at[pl.ds(0, _CH)], ob0, sw0).wait()
        pltpu.make_async_copy(g_hbm.at[pl.ds(0, _CH)], ob1, sw1).wait()

    @functools.partial(
        pl.kernel,
        out_type=jax.ShapeDtypeStruct((_NC, N, LATENT), F32),
        mesh=mesh,
        scratch_types=[
            pltpu.VMEM((_IDXROWS, _CH), jnp.int32),
            pltpu.VMEM((_CH, LATENT), F32),
            pltpu.VMEM((_CH, LATENT), F32),
            pltpu.VMEM_SHARED((N, LATENT), F32),
            pltpu.SemaphoreType.DMA,
            pltpu.SemaphoreType.DMA,
        ],
    )
    def scatter(e_hbm, rcv2_hbm, zz_hbm, out_hbm, idx_v, v0, v1, acc_sh,
                sv0, sv1):
        c = lax.axis_index("c")
        s = lax.axis_index("s")
        wid = s * _NC + c
        start, cnt = _worker_range(wid)
        base = (start // 8) * 8
        boff = start - base
        vals = (v0, v1)
        sv = (sv0, sv1)
        pltpu.sync_copy(rcv2_hbm.at[pl.ds(base, _IDXROWS)], idx_v)
        rbase = s * _ROWS_PER_SUB
        pltpu.sync_copy(zz_hbm.at[pl.ds(rbase, _ROWS_PER_SUB)],
                        acc_sh.at[pl.ds(rbase, _ROWS_PER_SUB)])

        @pl.when(s == _NS - 1)
        def _():
            pltpu.sync_copy(zz_hbm.at[pl.ds(_TAIL_BASE, _TAIL)],
                            acc_sh.at[pl.ds(_TAIL_BASE, _TAIL)])

        def fire(j, b):
            pltpu.async_copy(e_hbm.at[pl.ds((start + j) * _CH, _CH)],
                             vals[b], sv[b])

        fire(0, 0)
        fire(1, 1)
        plsc.subcore_barrier()

        @pl.loop(0, cnt)
        def _(j):
            for b in range(2):
                @pl.when(lax.rem(j, 2) == b)
                def _():
                    pltpu.make_async_copy(
                        e_hbm.at[pl.ds(0, _CH)], vals[b], sv[b]).wait()
                    pltpu.sync_copy(vals[b], acc_sh.at[idx_v.at[boff + j]],
                                    add=True)

                    @pl.when(j + 2 < cnt)
                    def _():
                        fire(j + 2, b)

        plsc.subcore_barrier()
        pltpu.sync_copy(acc_sh.at[pl.ds(rbase, _ROWS_PER_SUB)],
                        out_hbm.at[c, pl.ds(rbase, _ROWS_PER_SUB)])

        @pl.when(s == _NS - 1)
        def _():
            pltpu.sync_copy(acc_sh.at[pl.ds(_TAIL_BASE, _TAIL)],
                            out_hbm.at[c, pl.ds(_TAIL_BASE, _TAIL)])

    return gather, scatter


def _sc_gather(sp, rp, snd2, rcv2):
    return _sc_kernels()[0](sp, rp, snd2, rcv2)


def _sc_scatter(edges, rcv2, zz):
    return _sc_kernels()[1](edges, rcv2, zz)


def _pad2d_halves(idx):
    idx2 = idx.reshape(_NCHUNKS, _CH)
    pad = jnp.zeros((_PAD_CHUNKS - _NCH_H, _CH), jnp.int32)
    return (jnp.concatenate([idx2[:_NCH_H], pad], axis=0),
            jnp.concatenate([idx2[_NCH_H:], pad], axis=0))


def kernel(abs_pos, vel_hist, rel_disp, rel_dist, senders, receivers, tag,
           vel_mean, vel_std, acc_mean, acc_std, params):
    snd2a, snd2b = _pad2d_halves(senders.astype(jnp.int32))
    rcv2a, rcv2b = _pad2d_halves(receivers.astype(jnp.int32))
    tag3 = tag.astype(jnp.int32).reshape(N // NB, 1, NB)
    ef = jnp.concatenate([rel_disp, rel_dist], axis=-1)
    zz = jnp.zeros((N, LATENT), F32)

    def row(v):
        return v.reshape(1, -1)

    en = params["enc_node"]
    w0 = en[0]["w"]
    w0v, w0t = w0[: (HIST - 1) * DIM], w0[(HIST - 1) * DIM:]
    ew = params["embed"] @ w0t
    ee = params["enc_edge"]

    proc = params["proc"]

    def edge_w(t):
        l = proc[t]["edge_mlp"]
        w = l[0]["w"]
        return (w[:LATENT], w[LATENT:2 * LATENT], w[2 * LATENT:],
                row(l[0]["b"]), l[1]["w"], row(l[1]["b"]), l[2]["w"],
                row(l[2]["b"]), row(proc[t]["edge_ln"]["scale"]),
                row(proc[t]["edge_ln"]["offset"]))

    def node_w(t):
        l = proc[t]["node_mlp"]
        w = l[0]["w"]
        return (w[:LATENT], w[LATENT:], row(l[0]["b"]), l[1]["w"],
                row(l[1]["b"]), l[2]["w"], row(l[2]["b"]),
                row(proc[t]["node_ln"]["scale"]),
                row(proc[t]["node_ln"]["offset"]))

    we0, ws0, wr0 = edge_w(0)[:3]

    nodes, sp, rp = _node_encode(
        vel_hist, tag3, w0v, ew, row(en[0]["b"]), en[1]["w"], row(en[1]["b"]),
        en[2]["w"], row(en[2]["b"]), row(params["enc_node_ln"]["scale"]),
        row(params["enc_node_ln"]["offset"]), ws0, wr0)

    enc_edge_args = (
        ee[0]["w"], row(ee[0]["b"]), ee[1]["w"], row(ee[1]["b"]),
        ee[2]["w"], row(ee[2]["b"]), row(params["enc_edge_ln"]["scale"]),
        row(params["enc_edge_ln"]["offset"]))
    e_a = _edge_encode(ef[:_EH], *enc_edge_args)
    e_b = _edge_encode(ef[_EH:], *enc_edge_args)

    n_steps = len(proc)
    for t in range(n_steps):
        we, ws, wr, eb0, ew1, eb1, ew2, eb2, esc, eof = edge_w(t)
        ew_args = (we, eb0, ew1, eb1, ew2, eb2, esc, eof)
        g_a = _sc_gather(sp, rp, snd2a, rcv2a)
        g_b = _sc_gather(sp, rp, snd2b, rcv2b)
        e_a = _edge_update(e_a, g_a, *ew_args)
        agg_a = _sc_scatter(e_a, rcv2a, zz)
        e_b = _edge_update(e_b, g_b, *ew_args)
        agg_b = _sc_scatter(e_b, rcv2b, zz)
        wn, wa, nb0, nw1, nb1, nw2, nb2, nsc, nof = node_w(t)
        last = t == n_steps - 1
        ws_n, wr_n = (ws, wr) if last else edge_w(t + 1)[1:3]
        nodes, sp, rp = _node_update(
            nodes, agg_a[0], agg_a[1], agg_b[0], agg_b[1], wn, wa, nb0,
            nw1, nb1, nw2, nb2, nsc, nof, ws_n, wr_n, with_proj=not last)

    stats = jnp.zeros((8, LATENT), F32)
    stats = stats.at[0, :DIM].set(vel_mean).at[1, :DIM].set(vel_std)
    stats = stats.at[2, :DIM].set(acc_mean).at[3, :DIM].set(acc_std)
    dec = params["dec"]
    return _decode(
        nodes, abs_pos[:, -1], vel_hist[:, -DIM:], tag3, stats,
        dec[0]["w"], row(dec[0]["b"]), dec[1]["w"], row(dec[1]["b"]),
        dec[2]["w"], row(dec[2]["b"]))


# speedup vs baseline: 5.4023x; 1.0357x over previous
import functools

import jax
import jax.numpy as jnp
from jax import lax
from jax.experimental import pallas as pl
from jax.experimental.pallas import tpu as pltpu
from jax.experimental.pallas import tpu_sc as plsc

N = 10000
E = 160000
DIM = 3
HIST = 6
LATENT = 128
NUM_TYPES = 9
DT = 0.0025

NB = 2000
EB = 2000
F32 = jnp.float32

_NC = 2
_NS = 16
_NW = _NC * _NS
_CH = 128
_NCHUNKS = E // _CH
_ROWS_PER_SUB = 624
_TAIL_BASE = _ROWS_PER_SUB * _NS
_TAIL = N - _TAIL_BASE


def _ln(x, scale, offset):
    m = jnp.mean(x, axis=-1, keepdims=True)
    v = jnp.mean((x - m) ** 2, axis=-1, keepdims=True)
    return (x - m) / jnp.sqrt(v + 1e-6) * scale + offset


def _dot(a, b):
    return jnp.dot(a, b, preferred_element_type=F32)


def _node_enc_body(vh_ref, tag_ref, w0v_ref, ew_ref, b0_ref, w1_ref, b1_ref,
                   w2_ref, b2_ref, sc_ref, of_ref, ws_ref, wr_ref,
                   nodes_ref, sp_ref, rp_ref):
    tag = tag_ref[0, 0, :]
    oh = (tag[:, None] == lax.broadcasted_iota(jnp.int32, (NB, NUM_TYPES), 1)
          ).astype(F32)
    h = _dot(vh_ref[...], w0v_ref[...]) + _dot(oh, ew_ref[...]) + b0_ref[...]
    h = jnp.maximum(h, 0.0)
    h = jnp.maximum(_dot(h, w1_ref[...]) + b1_ref[...], 0.0)
    h = _dot(h, w2_ref[...]) + b2_ref[...]
    n = _ln(h, sc_ref[...], of_ref[...])
    nodes_ref[...] = n
    sp_ref[...] = _dot(n, ws_ref[...])
    rp_ref[...] = _dot(n, wr_ref[...])


def _edge_enc_body(ef_ref, w0_ref, b0_ref, w1_ref, b1_ref, w2_ref, b2_ref,
                   sc_ref, of_ref, out_ref):
    h = jnp.maximum(_dot(ef_ref[...], w0_ref[...]) + b0_ref[...], 0.0)
    h = jnp.maximum(_dot(h, w1_ref[...]) + b1_ref[...], 0.0)
    h = _dot(h, w2_ref[...]) + b2_ref[...]
    out_ref[...] = _ln(h, sc_ref[...], of_ref[...])


def _edge_upd_body(e_ref, g_ref, we_ref, b0_ref, w1_ref, b1_ref,
                   w2_ref, b2_ref, sc_ref, of_ref, out_ref):
    e = e_ref[...]
    h = _dot(e, we_ref[...]) + g_ref[...] + b0_ref[...]
    h = jnp.maximum(h, 0.0)
    h = jnp.maximum(_dot(h, w1_ref[...]) + b1_ref[...], 0.0)
    h = _dot(h, w2_ref[...]) + b2_ref[...]
    out_ref[...] = e + _ln(h, sc_ref[...], of_ref[...])


def _node_upd_body(n_ref, a0_ref, a1_ref, a2_ref, a3_ref, wn_ref, wa_ref,
                   b0_ref, w1_ref, b1_ref, w2_ref, b2_ref, sc_ref, of_ref,
                   ws_ref, wr_ref, out_ref, sp_ref, rp_ref, *, with_proj):
    n = n_ref[...]
    agg = (a0_ref[...] + a1_ref[...]) + (a2_ref[...] + a3_ref[...])
    h = _dot(n, wn_ref[...]) + _dot(agg, wa_ref[...]) + b0_ref[...]
    h = jnp.maximum(h, 0.0)
    h = jnp.maximum(_dot(h, w1_ref[...]) + b1_ref[...], 0.0)
    h = _dot(h, w2_ref[...]) + b2_ref[...]
    nn = n + _ln(h, sc_ref[...], of_ref[...])
    out_ref[...] = nn
    if with_proj:
        sp_ref[...] = _dot(nn, ws_ref[...])
        rp_ref[...] = _dot(nn, wr_ref[...])


def _dec_body(n_ref, r0_ref, lv_ref, tag_ref, st_ref, w0_ref, b0_ref, w1_ref,
              b1_ref, w2_ref, b2_ref, out_ref):
    h = jnp.maximum(_dot(n_ref[...], w0_ref[...]) + b0_ref[...], 0.0)
    h = jnp.maximum(_dot(h, w1_ref[...]) + b1_ref[...], 0.0)
    acc = _dot(h, w2_ref[...]) + b2_ref[...]
    st = st_ref[...]
    vm, vs = st[0:1, 0:DIM], st[1:2, 0:DIM]
    am, asd = st[2:3, 0:DIM], st[3:4, 0:DIM]
    r0 = r0_ref[...]
    u0 = (lv_ref[...] * vs + vm) / DT
    a = (acc * asd + am) / (DT * DT)
    u = u0 + DT * a
    r = r0 + DT * u
    r = r - jnp.floor(r)
    tag2d = tag_ref[0, 0, :][:, None]
    wall = jnp.where(tag2d == 3, 1.0, 0.0)
    r = wall * r0 + (1.0 - wall) * (r - jnp.floor(r))
    d = r - r0
    d = d - jnp.round(d)
    u2 = d / DT
    a2 = (u2 - u0) / DT
    out_ref[...] = (a2 * (DT * DT) - am) / asd


def _full(shape):
    return pl.BlockSpec(shape, lambda i: tuple(0 for _ in shape))


def _rowblk(cols, rows=NB):
    return pl.BlockSpec((rows, cols), lambda i: (i, 0))


_TAG_SPEC = pl.BlockSpec((1, 1, NB), lambda i: (i, 0, 0))


def _w_specs(*shapes):
    return [_full(s) for s in shapes]


def _node_encode(vh, tag3, w0v, ew, b0, w1, b1, w2, b2, sc, of, ws, wr):
    grid = (N // NB,)
    out_shape = [jax.ShapeDtypeStruct((N, LATENT), F32)] * 3
    return pl.pallas_call(
        _node_enc_body,
        grid=grid,
        in_specs=[_rowblk((HIST - 1) * DIM), _TAG_SPEC] + _w_specs(
            w0v.shape, ew.shape, b0.shape, w1.shape, b1.shape, w2.shape,
            b2.shape, sc.shape, of.shape, ws.shape, wr.shape),
        out_specs=[_rowblk(LATENT)] * 3,
        out_shape=out_shape,
    )(vh, tag3, w0v, ew, b0, w1, b1, w2, b2, sc, of, ws, wr)


def _edge_encode(ef, w0, b0, w1, b1, w2, b2, sc, of):
    grid = (_EH // EB,)
    return pl.pallas_call(
        _edge_enc_body,
        grid=grid,
        in_specs=[_rowblk(DIM + 1, EB)] + _w_specs(
            w0.shape, b0.shape, w1.shape, b1.shape, w2.shape, b2.shape,
            sc.shape, of.shape),
        out_specs=_rowblk(LATENT, EB),
        out_shape=jax.ShapeDtypeStruct((_EH, LATENT), F32),
    )(ef, w0, b0, w1, b1, w2, b2, sc, of)


def _edge_update(e, g, we, b0, w1, b1, w2, b2, sc, of):
    grid = (_EH // EB,)
    return pl.pallas_call(
        _edge_upd_body,
        grid=grid,
        in_specs=[_rowblk(LATENT, EB)] * 2 + _w_specs(
            we.shape, b0.shape, w1.shape, b1.shape, w2.shape, b2.shape,
            sc.shape, of.shape),
        out_specs=_rowblk(LATENT, EB),
        out_shape=jax.ShapeDtypeStruct((_EH, LATENT), F32),
    )(e, g, we, b0, w1, b1, w2, b2, sc, of)


def _node_update(n, a0, a1, a2, a3, wn, wa, b0, w1, b1, w2, b2, sc, of,
                 ws, wr, with_proj):
    grid = (N // NB,)
    nout = 3 if with_proj else 1
    body = functools.partial(_node_upd_body, with_proj=with_proj)
    if not with_proj:
        def body(*refs):
            _node_upd_body(*refs[:16], refs[16], None, None, with_proj=False)
    res = pl.pallas_call(
        body,
        grid=grid,
        in_specs=[_rowblk(LATENT)] * 5 + _w_specs(
            wn.shape, wa.shape, b0.shape, w1.shape, b1.shape, w2.shape,
            b2.shape, sc.shape, of.shape, ws.shape, wr.shape),
        out_specs=[_rowblk(LATENT)] * nout,
        out_shape=[jax.ShapeDtypeStruct((N, LATENT), F32)] * nout,
    )(n, a0, a1, a2, a3, wn, wa, b0, w1, b1, w2, b2, sc, of, ws, wr)
    if with_proj:
        return res
    return res[0], None, None


def _decode(n, r0, lv, tag3, stats, w0, b0, w1, b1, w2, b2):
    grid = (N // NB,)
    return pl.pallas_call(
        _dec_body,
        grid=grid,
        in_specs=[_rowblk(LATENT), _rowblk(DIM), _rowblk(DIM), _TAG_SPEC]
        + _w_specs(stats.shape, w0.shape, b0.shape, w1.shape, b1.shape,
                   w2.shape, b2.shape),
        out_specs=_rowblk(DIM),
        out_shape=jax.ShapeDtypeStruct((N, DIM), F32),
    )(n, r0, lv, tag3, stats, w0, b0, w1, b1, w2, b2)


_EH = E // 2
_NCH_H = _EH // _CH
_PAD_CHUNKS = 640
_IDXROWS = 32


def _worker_range(wid):
    q, r = divmod(_NCH_H, _NW)
    start = wid * q + jnp.minimum(wid, r)
    cnt = q + (wid < r).astype(jnp.int32)
    return start, cnt


@functools.cache
def _sc_kernels():
    mesh = plsc.VectorSubcoreMesh(core_axis_name="c", subcore_axis_name="s",
                                  num_cores=_NC, num_subcores=_NS)

    @functools.partial(
        pl.kernel,
        out_type=jax.ShapeDtypeStruct((_EH, LATENT), F32),
        mesh=mesh,
        scratch_types=[
            pltpu.VMEM((_IDXROWS, _CH), jnp.int32),
            pltpu.VMEM((_IDXROWS, _CH), jnp.int32),
            pltpu.VMEM((_CH, LATENT), F32),
            pltpu.VMEM((_CH, LATENT), F32),
            pltpu.VMEM((_CH, LATENT), F32),
            pltpu.VMEM((_CH, LATENT), F32),
            pltpu.VMEM((_CH, LATENT), F32),
            pltpu.VMEM((_CH, LATENT), F32),
            pltpu.SemaphoreType.DMA,
            pltpu.SemaphoreType.DMA,
            pltpu.SemaphoreType.DMA,
            pltpu.SemaphoreType.DMA,
        ],
    )
    def gather(sp_hbm, rp_hbm, snd2_hbm, rcv2_hbm, g_hbm,
               idx_s, idx_r, rs0, rs1, rr0, rr1, ob0, ob1,
               sg0, sg1, sw0, sw1):
        wid = lax.axis_index("s") * _NC + lax.axis_index("c")
        start, cnt = _worker_range(wid)
        base = (start // 8) * 8
        boff = start - base
        rs = (rs0, rs1)
        rr = (rr0, rr1)
        ob = (ob0, ob1)
        sg = (sg0, sg1)
        sw = (sw0, sw1)

        pltpu.sync_copy(snd2_hbm.at[pl.ds(base, _IDXROWS)], idx_s)
        pltpu.sync_copy(rcv2_hbm.at[pl.ds(base, _IDXROWS)], idx_r)

        def fire(j, b):
            pltpu.async_copy(sp_hbm.at[idx_s.at[boff + j]], rs[b], sg[b])
            pltpu.async_copy(rp_hbm.at[idx_r.at[boff + j]], rr[b], sg[b])

        fire(0, 0)
        fire(1, 1)

        @pl.loop(0, cnt)
        def _(j):
            for b in range(2):
                @pl.when(lax.rem(j, 2) == b)
                def _():
                    @pl.when(j >= 2)
                    def _():
                        pltpu.make_async_copy(
                            g_hbm.at[pl.ds(0, _CH)], ob[b], sw[b]).wait()
                    pltpu.make_async_copy(
                        sp_hbm.at[pl.ds(0, _CH)], rs[b], sg[b]).wait()
                    pltpu.make_async_copy(
                        sp_hbm.at[pl.ds(0, _CH)], rr[b], sg[b]).wait()

                    @pl.loop(0, _CH)
                    def _(r):
                        for k in range(LATENT // 16):
                            sl = pl.ds(k * 16, 16)
                            ob[b][r, sl] = rs[b][r, sl] + rr[b][r, sl]

                    off = (start + j) * _CH
                    pltpu.async_copy(ob[b], g_hbm.at[pl.ds(off, _CH)], sw[b])

                    @pl.when(j + 2 < cnt)
                    def _():
                        fire(j + 2, b)

        pltpu.make_async_copy(g_hbm.at[pl.ds(0, _CH)], ob0, sw0).wait()
        pltpu.make_async_copy(g_hbm.at[pl.ds(0, _CH)], ob1, sw1).wait()

    @functools.partial(
        pl.kernel,
        out_type=[jax.ShapeDtypeStruct((N, LATENT), F32)] * _NC,
        mesh=mesh,
        scratch_types=[
            pltpu.VMEM((_IDXROWS, _CH), jnp.int32),
            pltpu.VMEM((_CH, LATENT), F32),
            pltpu.VMEM((_CH, LATENT), F32),
            pltpu.VMEM_SHARED((N, LATENT), F32),
            pltpu.SemaphoreType.DMA,
            pltpu.SemaphoreType.DMA,
        ],
    )
    def scatter(e_hbm, rcv2_hbm, zz_hbm, out0_hbm, out1_hbm, idx_v, v0, v1,
                acc_sh, sv0, sv1):
        c = lax.axis_index("c")
        s = lax.axis_index("s")
        wid = s * _NC + c
        start, cnt = _worker_range(wid)
        base = (start // 8) * 8
        boff = start - base
        vals = (v0, v1)
        sv = (sv0, sv1)
        pltpu.sync_copy(rcv2_hbm.at[pl.ds(base, _IDXROWS)], idx_v)
        rbase = s * _ROWS_PER_SUB
        pltpu.sync_copy(zz_hbm.at[pl.ds(rbase, _ROWS_PER_SUB)],
                        acc_sh.at[pl.ds(rbase, _ROWS_PER_SUB)])

        @pl.when(s == _NS - 1)
        def _():
            pltpu.sync_copy(zz_hbm.at[pl.ds(_TAIL_BASE, _TAIL)],
                            acc_sh.at[pl.ds(_TAIL_BASE, _TAIL)])

        def fire(j, b):
            pltpu.async_copy(e_hbm.at[pl.ds((start + j) * _CH, _CH)],
                             vals[b], sv[b])

        fire(0, 0)
        fire(1, 1)
        plsc.subcore_barrier()

        @pl.loop(0, cnt)
        def _(j):
            for b in range(2):
                @pl.when(lax.rem(j, 2) == b)
                def _():
                    pltpu.make_async_copy(
                        e_hbm.at[pl.ds(0, _CH)], vals[b], sv[b]).wait()
                    pltpu.sync_copy(vals[b], acc_sh.at[idx_v.at[boff + j]],
                                    add=True)

                    @pl.when(j + 2 < cnt)
                    def _():
                        fire(j + 2, b)

        plsc.subcore_barrier()
        for cc, out_hbm in enumerate((out0_hbm, out1_hbm)):
            @pl.when(c == cc)
            def _():
                pltpu.sync_copy(acc_sh.at[pl.ds(rbase, _ROWS_PER_SUB)],
                                out_hbm.at[pl.ds(rbase, _ROWS_PER_SUB)])

                @pl.when(s == _NS - 1)
                def _():
                    pltpu.sync_copy(acc_sh.at[pl.ds(_TAIL_BASE, _TAIL)],
                                    out_hbm.at[pl.ds(_TAIL_BASE, _TAIL)])

    return gather, scatter


def _sc_gather(sp, rp, snd2, rcv2):
    return _sc_kernels()[0](sp, rp, snd2, rcv2)


def _sc_scatter(edges, rcv2, zz):
    return _sc_kernels()[1](edges, rcv2, zz)


def _pad2d_halves(idx):
    idx2 = idx.reshape(_NCHUNKS, _CH)
    pad = jnp.zeros((_PAD_CHUNKS - _NCH_H, _CH), jnp.int32)
    return (jnp.concatenate([idx2[:_NCH_H], pad], axis=0),
            jnp.concatenate([idx2[_NCH_H:], pad], axis=0))


def kernel(abs_pos, vel_hist, rel_disp, rel_dist, senders, receivers, tag,
           vel_mean, vel_std, acc_mean, acc_std, params):
    snd2a, snd2b = _pad2d_halves(senders.astype(jnp.int32))
    rcv2a, rcv2b = _pad2d_halves(receivers.astype(jnp.int32))
    tag3 = tag.astype(jnp.int32).reshape(N // NB, 1, NB)
    ef = jnp.concatenate([rel_disp, rel_dist], axis=-1)
    zz = jnp.zeros((N, LATENT), F32)

    def row(v):
        return v.reshape(1, -1)

    en = params["enc_node"]
    w0 = en[0]["w"]
    w0v, w0t = w0[: (HIST - 1) * DIM], w0[(HIST - 1) * DIM:]
    ew = params["embed"] @ w0t
    ee = params["enc_edge"]

    proc = params["proc"]

    def edge_w(t):
        l = proc[t]["edge_mlp"]
        w = l[0]["w"]
        return (w[:LATENT], w[LATENT:2 * LATENT], w[2 * LATENT:],
                row(l[0]["b"]), l[1]["w"], row(l[1]["b"]), l[2]["w"],
                row(l[2]["b"]), row(proc[t]["edge_ln"]["scale"]),
                row(proc[t]["edge_ln"]["offset"]))

    def node_w(t):
        l = proc[t]["node_mlp"]
        w = l[0]["w"]
        return (w[:LATENT], w[LATENT:], row(l[0]["b"]), l[1]["w"],
                row(l[1]["b"]), l[2]["w"], row(l[2]["b"]),
                row(proc[t]["node_ln"]["scale"]),
                row(proc[t]["node_ln"]["offset"]))

    we0, ws0, wr0 = edge_w(0)[:3]

    nodes, sp, rp = _node_encode(
        vel_hist, tag3, w0v, ew, row(en[0]["b"]), en[1]["w"], row(en[1]["b"]),
        en[2]["w"], row(en[2]["b"]), row(params["enc_node_ln"]["scale"]),
        row(params["enc_node_ln"]["offset"]), ws0, wr0)

    enc_edge_args = (
        ee[0]["w"], row(ee[0]["b"]), ee[1]["w"], row(ee[1]["b"]),
        ee[2]["w"], row(ee[2]["b"]), row(params["enc_edge_ln"]["scale"]),
        row(params["enc_edge_ln"]["offset"]))
    e_a = _edge_encode(ef[:_EH], *enc_edge_args)
    e_b = _edge_encode(ef[_EH:], *enc_edge_args)

    n_steps = len(proc)
    for t in range(n_steps):
        we, ws, wr, eb0, ew1, eb1, ew2, eb2, esc, eof = edge_w(t)
        ew_args = (we, eb0, ew1, eb1, ew2, eb2, esc, eof)
        g_a = _sc_gather(sp, rp, snd2a, rcv2a)
        g_b = _sc_gather(sp, rp, snd2b, rcv2b)
        e_a = _edge_update(e_a, g_a, *ew_args)
        agg_a0, agg_a1 = _sc_scatter(e_a, rcv2a, zz)
        e_b = _edge_update(e_b, g_b, *ew_args)
        agg_b0, agg_b1 = _sc_scatter(e_b, rcv2b, zz)
        wn, wa, nb0, nw1, nb1, nw2, nb2, nsc, nof = node_w(t)
        last = t == n_steps - 1
        ws_n, wr_n = (ws, wr) if last else edge_w(t + 1)[1:3]
        nodes, sp, rp = _node_update(
            nodes, agg_a0, agg_a1, agg_b0, agg_b1, wn, wa, nb0,
            nw1, nb1, nw2, nb2, nsc, nof, ws_n, wr_n, with_proj=not last)

    stats = jnp.zeros((8, LATENT), F32)
    stats = stats.at[0, :DIM].set(vel_mean).at[1, :DIM].set(vel_std)
    stats = stats.at[2, :DIM].set(acc_mean).at[3, :DIM].set(acc_std)
    dec = params["dec"]
    return _decode(
        nodes, abs_pos[:, -1], vel_hist[:, -DIM:], tag3, stats,
        dec[0]["w"], row(dec[0]["b"]), dec[1]["w"], row(dec[1]["b"]),
        dec[2]["w"], row(dec[2]["b"]))
